# baseline (speedup 1.0000x reference)
import jax
import jax.numpy as jnp
from jax import lax
from jax.experimental import pallas as pl
from jax.experimental.pallas import tpu as pltpu
from jax.experimental.pallas import tpu_sc as plsc

H = 2048
W = 2048
C = 3
N = 1048576

NUM_WORKERS = 32
PTS_PER_TILE = N // NUM_WORKERS
CHUNK = 128
G = PTS_PER_TILE // CHUNK
L = 16
D = 16


def _body(xs_hbm, table_hbm, out_hbm,
          xs_all,
          idx0, wx0, wy0, gq0,
          idx1, wx1, wy1, gq1,
          obuf, sem0, sem1):
  wid = lax.axis_index("s") * 2 + lax.axis_index("c")
  lane = lax.broadcasted_iota(jnp.int32, (L,), 0)
  slots = (
      (idx0, wx0, wy0, gq0, sem0),
      (idx1, wx1, wy1, gq1, sem1),
  )

  pltpu.sync_copy(xs_hbm.at[pl.ds(2 * wid * PTS_PER_TILE, 2 * PTS_PER_TILE)],
                  xs_all)

  def pass1(g, slot):
    idx_ref, wx_ref, wy_ref, _, _ = slot
    for q in range(CHUNK // L):
      pbase = q * L
      ex = 2 * (g * CHUNK + pbase + lane)
      px = plsc.load_gather(xs_all, [ex])
      py = plsc.load_gather(xs_all, [ex + 1])
      sx = px * jnp.float32(W)
      sy = py * jnp.float32(H)
      ix = sx.astype(jnp.int32)
      iy = sy.astype(jnp.int32)
      wx = sx - ix.astype(jnp.float32)
      wy = sy - iy.astype(jnp.float32)
      x0 = jnp.minimum(jnp.maximum(ix, 0), W - 1)
      y0 = jnp.minimum(jnp.maximum(iy, 0), H - 1)
      sl = pl.ds(pbase, L)
      idx_ref[sl] = y0 * W + x0
      wx_ref[sl] = wx
      wy_ref[sl] = wy

  def fire(slot):
    idx_ref, _, _, gq, sem = slot
    pltpu.async_copy(table_hbm.at[idx_ref], gq, sem)

  def drain(slot):
    idx_ref, _, _, gq, sem = slot
    pltpu.make_async_copy(table_hbm.at[idx_ref], gq, sem).wait()

  def combine(slot, b):
    _, wx_ref, wy_ref, gq, _ = slot
    for q in range(CHUNK // L):
      pbase = q * L
      sl = pl.ds(pbase, L)
      wx = wx_ref[sl]
      wy = wy_ref[sl]
      prow = pbase + lane
      obase = b * (C * CHUNK)
      for c in range(C):
        t0 = plsc.load_gather(gq, [prow, jnp.full((L,), c, jnp.int32)])
        t1 = plsc.load_gather(gq, [prow, jnp.full((L,), c + C, jnp.int32)])
        b0 = plsc.load_gather(gq, [prow, jnp.full((L,), c + 2 * C, jnp.int32)])
        b1 = plsc.load_gather(gq, [prow, jnp.full((L,), c + 3 * C, jnp.int32)])
        top = t0 + wx * (t1 - t0)
        bot = b0 + wx * (b1 - b0)
        o = top + wy * (bot - top)
        plsc.store_scatter(obuf, [obase + 3 * prow + c], o)

  for b in (0, 1):
    pass1(jnp.int32(b), slots[b])
    fire(slots[b])

  def body(i, carry):
    for b in (0, 1):
      g = 2 * i + b
      drain(slots[b])
      combine(slots[b], b)
      gn = g + 2
      gn = jnp.where(gn >= G, gn - G, gn)
      pass1(gn, slots[b])
      fire(slots[b])
    pltpu.sync_copy(
        obuf, out_hbm.at[pl.ds(3 * (wid * PTS_PER_TILE + 2 * i * CHUNK),
                               2 * C * CHUNK)])
    return carry

  lax.fori_loop(0, G // 2, body, 0)
  drain(slots[0])
  drain(slots[1])


@jax.jit
def _run(xs_flat, table):
  mesh = plsc.VectorSubcoreMesh(core_axis_name="c", subcore_axis_name="s")
  slot_types = [
      pltpu.VMEM((CHUNK,), jnp.int32),
      pltpu.VMEM((CHUNK,), jnp.float32),
      pltpu.VMEM((CHUNK,), jnp.float32),
      pltpu.VMEM((CHUNK, D), jnp.float32),
  ]
  kern = pl.kernel(
      _body,
      out_type=jax.ShapeDtypeStruct((N * C,), jnp.float32),
      mesh=mesh,
      compiler_params=pltpu.CompilerParams(
          needs_layout_passes=False, use_tc_tiling_on_sc=False),
      scratch_types=(
          [pltpu.VMEM((2 * PTS_PER_TILE,), jnp.float32)]
          + slot_types + slot_types
          + [pltpu.VMEM((2 * C * CHUNK,), jnp.float32),
             pltpu.SemaphoreType.DMA,
             pltpu.SemaphoreType.DMA]
      ),
  )
  return kern(xs_flat, table)


def kernel(xs, data):
  rx = jnp.concatenate([data[:, 1:], data[:, -1:]], axis=1)
  ry = jnp.concatenate([data[1:], data[-1:]], axis=0)
  rxy = jnp.concatenate([ry[:, 1:], ry[:, -1:]], axis=1)
  quad = jnp.concatenate(
      [data, rx, ry, rxy, jnp.zeros((H, W, D - 4 * C), jnp.float32)], axis=2)
  out_flat = _run(xs.reshape(-1), quad.reshape(H * W, D))
  return out_flat.reshape(N, C)


# speedup vs baseline: 15.2096x; 15.2096x over previous
import jax
import jax.numpy as jnp
from jax import lax
from jax.experimental import pallas as pl
from jax.experimental.pallas import tpu as pltpu
from jax.experimental.pallas import tpu_sc as plsc

H = 2048
W = 2048
C = 3
N = 1048576

NUM_WORKERS = 32
PTS_PER_TILE = N // NUM_WORKERS
CHUNK = 128
G = PTS_PER_TILE // CHUNK
L = 16
D = 8


def _body(xs_hbm, table_hbm, out_hbm,
          xs_all,
          idx_top0, idx_bot0, wx0, wy0, gt0, gb0,
          idx_top1, idx_bot1, wx1, wy1, gt1, gb1,
          obuf, sem0, sem1):
  wid = lax.axis_index("s") * 2 + lax.axis_index("c")
  lane = lax.broadcasted_iota(jnp.int32, (L,), 0)
  slots = (
      (idx_top0, idx_bot0, wx0, wy0, gt0, gb0, sem0),
      (idx_top1, idx_bot1, wx1, wy1, gt1, gb1, sem1),
  )

  pltpu.sync_copy(xs_hbm.at[pl.ds(2 * wid * PTS_PER_TILE, 2 * PTS_PER_TILE)],
                  xs_all)

  def pass1(g, slot):
    idx_top, idx_bot, wx_ref, wy_ref, *_ = slot
    for q in range(CHUNK // L):
      pbase = q * L
      ex = 2 * (g * CHUNK + pbase + lane)
      px = plsc.load_gather(xs_all, [ex])
      py = plsc.load_gather(xs_all, [ex + 1])
      sx = px * jnp.float32(W)
      sy = py * jnp.float32(H)
      ix = sx.astype(jnp.int32)
      iy = sy.astype(jnp.int32)
      wx = sx - ix.astype(jnp.float32)
      wy = sy - iy.astype(jnp.float32)
      x0 = jnp.minimum(jnp.maximum(ix, 0), W - 1)
      y0 = jnp.minimum(jnp.maximum(iy, 0), H - 1)
      y1 = jnp.minimum(y0 + 1, H - 1)
      wx = jnp.where(x0 >= W - 1, jnp.float32(0.0), wx)
      sl = pl.ds(pbase, L)
      idx_top[sl] = y0 * W + x0
      idx_bot[sl] = y1 * W + x0
      wx_ref[sl] = wx
      wy_ref[sl] = wy

  def fire(slot):
    idx_top, idx_bot, _, _, gt, gb, sem = slot
    pltpu.async_copy(table_hbm.at[idx_top], gt, sem)
    pltpu.async_copy(table_hbm.at[idx_bot], gb, sem)

  def drain(slot):
    idx_top, idx_bot, _, _, gt, gb, sem = slot
    pltpu.make_async_copy(table_hbm.at[idx_top], gt, sem).wait()
    pltpu.make_async_copy(table_hbm.at[idx_bot], gb, sem).wait()

  def combine(slot, b):
    _, _, wx_ref, wy_ref, gt, gb, _ = slot
    for q in range(CHUNK // L):
      pbase = q * L
      sl = pl.ds(pbase, L)
      wx = wx_ref[sl]
      wy = wy_ref[sl]
      prow = pbase + lane
      obase = b * (C * CHUNK)
      for c in range(C):
        c0col = jnp.full((L,), c, jnp.int32)
        c1col = jnp.full((L,), c + C, jnp.int32)
        t0 = plsc.load_gather(gt, [prow, c0col])
        t1 = plsc.load_gather(gt, [prow, c1col])
        b0 = plsc.load_gather(gb, [prow, c0col])
        b1 = plsc.load_gather(gb, [prow, c1col])
        top = t0 + wx * (t1 - t0)
        bot = b0 + wx * (b1 - b0)
        o = top + wy * (bot - top)
        plsc.store_scatter(obuf, [obase + 3 * prow + c], o)

  for b in (0, 1):
    pass1(jnp.int32(b), slots[b])
    fire(slots[b])

  def body(i, carry):
    for b in (0, 1):
      g = 2 * i + b
      drain(slots[b])
      combine(slots[b], b)
      gn = g + 2
      gn = jnp.where(gn >= G, gn - G, gn)
      pass1(gn, slots[b])
      fire(slots[b])
    pltpu.sync_copy(
        obuf, out_hbm.at[pl.ds(3 * (wid * PTS_PER_TILE + 2 * i * CHUNK),
                               2 * C * CHUNK)])
    return carry

  lax.fori_loop(0, G // 2, body, 0)
  drain(slots[0])
  drain(slots[1])


@jax.jit
def _run(xs_flat, table):
  mesh = plsc.VectorSubcoreMesh(core_axis_name="c", subcore_axis_name="s")
  slot_types = [
      pltpu.VMEM((CHUNK,), jnp.int32),
      pltpu.VMEM((CHUNK,), jnp.int32),
      pltpu.VMEM((CHUNK,), jnp.float32),
      pltpu.VMEM((CHUNK,), jnp.float32),
      pltpu.VMEM((CHUNK, D), jnp.float32),
      pltpu.VMEM((CHUNK, D), jnp.float32),
  ]
  kern = pl.kernel(
      _body,
      out_type=jax.ShapeDtypeStruct((N * C,), jnp.float32),
      mesh=mesh,
      compiler_params=pltpu.CompilerParams(
          needs_layout_passes=False, use_tc_tiling_on_sc=False),
      scratch_types=(
          [pltpu.VMEM((2 * PTS_PER_TILE,), jnp.float32)]
          + slot_types + slot_types
          + [pltpu.VMEM((2 * C * CHUNK,), jnp.float32),
             pltpu.SemaphoreType.DMA,
             pltpu.SemaphoreType.DMA]
      ),
  )
  return kern(xs_flat, table)


def kernel(xs, data):
  rows = data.reshape(H * W, C)
  nxt = jnp.concatenate([rows[1:], rows[-1:]], axis=0)
  table = jnp.concatenate(
      [rows, nxt, jnp.zeros((H * W, D - 2 * C), jnp.float32)], axis=1)
  out_flat = _run(xs.reshape(-1), table)
  return out_flat.reshape(N, C)
